# baseline (speedup 1.0000x reference)
import functools
import jax
import jax.numpy as jnp
import numpy as np
from jax import lax
from jax.experimental import pallas as pl
from jax.experimental.pallas import tpu as pltpu
from jax.experimental.pallas import tpu_sc as plsc

_NR = 8
_NW = 8
_H = 48
_NB = _NR * _NW
_N = 2048
_FC_OUT = 128
_BI = 16
_BR = 256

_N_TC = 0
_N_SC = _N - _N_TC


def _pairwise_bins(ydash):
    r = jnp.linalg.norm(ydash[:, None, :] - ydash[None, :, :], axis=2)
    ring_ids = jnp.ceil((_NR - 1) * (jnp.log(r / 0.5) / 3.0))
    ring_ids = jnp.where(jnp.isneginf(ring_ids), 0.0, ring_ids)
    ring_ids = ring_ids.astype(jnp.int32)
    x_diff = ydash[:, 0] - ydash[:, 0][:, None]
    y_diff = ydash[:, 1] - ydash[:, 1][:, None]
    theta = jnp.arctan2(y_diff, x_diff)
    wedge_ids = theta * _NW / (2 * np.pi)
    wedge_ids = (wedge_ids + (_NW // 2 - 1)).astype(jnp.int32)
    fin = ring_ids * _NW + wedge_ids
    fin = jnp.where(fin < _NW, 0, fin)
    fin = jnp.where(fin >= _NW * _NW, 0, fin)
    return fin.astype(jnp.int32)


def _means_step(fin_ref, hid, m_ref):
    fin = fin_ref[...]
    k_iota = jax.lax.broadcasted_iota(jnp.int32, (_BI, _NB, _N), 1)
    onehot = (fin[:, None, :] == k_iota + _NW).astype(jnp.float32)
    onehot = onehot.reshape(_BI * _NB, _N)

    s = jnp.dot(onehot, hid[...], preferred_element_type=jnp.float32)
    cnt = s[:, _H:_H + 1]
    recip = 1.0 / jnp.maximum(cnt, 1.0)
    m_ref[...] = s[:, :_H] * recip


def _tc_means(fin_tc, hidden):
    hidden_aug = jnp.concatenate(
        [hidden, jnp.ones((_N, 1), hidden.dtype),
         jnp.zeros((_N, 64 - _H - 1), hidden.dtype)], axis=1)
    return pl.pallas_call(
        _means_step,
        grid=(_N_TC // _BI,),
        in_specs=[
            pl.BlockSpec((_BI, _N), lambda i: (i, 0)),
            pl.BlockSpec((_N, 64), lambda i: (0, 0)),
        ],
        out_specs=pl.BlockSpec((_BI * _NB, _H), lambda i: (i, 0)),
        out_shape=jax.ShapeDtypeStruct((_N_TC * _NB, _H), jnp.float32),
    )(fin_tc, hidden_aug)


def _recip_step(fin_ref, r_ref):
    fin = fin_ref[...]
    k_iota = jax.lax.broadcasted_iota(jnp.int32, (_BI, _NB, _N), 1)
    cnt = jnp.sum((fin[:, None, :] == k_iota + _NW).astype(jnp.float32),
                  axis=2)
    r_ref[...] = 1.0 / jnp.maximum(cnt, 1.0)


def _tc_recip(fin_sc, nsc):
    return pl.pallas_call(
        _recip_step,
        grid=(nsc // _BI,),
        in_specs=[pl.BlockSpec((_BI, _N), lambda i: (i, 0))],
        out_specs=pl.BlockSpec((_BI, _NB), lambda i: (i, 0)),
        out_shape=jax.ShapeDtypeStruct((nsc, _NB), jnp.float32),
    )(fin_sc)


def _make_sc_means(nsc):
    ag_per_w = nsc // 32
    mesh = plsc.VectorSubcoreMesh(core_axis_name="c", subcore_axis_name="s")

    @functools.partial(
        pl.kernel,
        mesh=mesh,
        compiler_params=pltpu.CompilerParams(use_tc_tiling_on_sc=False),
        out_type=jax.ShapeDtypeStruct((nsc * _NB, _H), jnp.float32),
        scratch_types=[
            pltpu.VMEM((_N * _H,), jnp.float32),
            pltpu.VMEM((_N,), jnp.int32),
            pltpu.VMEM((4, _NB * _H), jnp.float32),
            pltpu.VMEM((_NB, _H), jnp.float32),
            pltpu.VMEM((_NB,), jnp.float32),
            pltpu.SemaphoreType.DMA,
        ],
    )
    def sc_means(fin_hbm, hidden1d_hbm, recip_hbm, out_hbm,
                 hid_v, fin_v, acc_v, out_v, recip_v, sem):
        cid = lax.axis_index("c")
        sid = lax.axis_index("s")
        wid = cid * 16 + sid

        pltpu.sync_copy(hidden1d_hbm, hid_v)

        z16 = jnp.zeros((16,), jnp.float32)
        for row in range(_NB):
            for c3 in range(_H // 16):
                out_v[row, pl.ds(c3 * 16, 16)] = z16

        def agent_body(a, _):
            agent = wid * ag_per_w + a
            pltpu.sync_copy(fin_hbm.at[agent], fin_v)
            pltpu.sync_copy(recip_hbm.at[agent], recip_v)
            for p in range(4):
                for t in range(_NB * _H // 16):
                    acc_v[p, pl.ds(t * 16, 16)] = z16

            def j_body(g, _):
                j0 = g * 16
                idxv = fin_v[pl.ds(j0, 16)] * _H
                for u in range(16):
                    off = idxv[u]
                    src = (j0 + u) * _H
                    for c3 in range(_H // 16):
                        sl = pl.ds(off + c3 * 16, 16)
                        acc_v[u % 4, sl] = (
                            acc_v[u % 4, sl]
                            + hid_v[pl.ds(src + c3 * 16, 16)])
                return 0

            lax.fori_loop(0, _N // 16, j_body, 0)

            for k in range(_NW, _NB):
                slot = k - _NW
                rvec = recip_v[pl.ds((slot // 16) * 16, 16)]
                rv = lax.gather(
                    rvec, jnp.full((16, 1), slot % 16, jnp.int32),
                    lax.GatherDimensionNumbers(
                        offset_dims=(), collapsed_slice_dims=(0,),
                        start_index_map=(0,)),
                    slice_sizes=(1,),
                    mode=lax.GatherScatterMode.PROMISE_IN_BOUNDS)
                for c3 in range(_H // 16):
                    sl = pl.ds(k * _H + c3 * 16, 16)
                    tot = ((acc_v[0, sl] + acc_v[1, sl])
                           + (acc_v[2, sl] + acc_v[3, sl]))
                    out_v[k - _NW, pl.ds(c3 * 16, 16)] = tot * rv

            pltpu.sync_copy(out_v, out_hbm.at[pl.ds(agent * _NB, _NB)])
            return 0

        lax.fori_loop(0, ag_per_w, agent_body, 0)

    return sc_means


def _fc_step(m, Wt, b, out_ref):
    o = jnp.dot(m[...], Wt[...], preferred_element_type=jnp.float32) + b[...]
    out_ref[...] = jnp.maximum(o, 0.0)


def kernel(y_pred, x_start, hidden, W, b):
    del x_start
    fin = _pairwise_bins(jax.lax.stop_gradient(y_pred))

    parts = []
    if _N_TC:
        parts.append(_tc_means(fin[:_N_TC], hidden))
    if _N_SC:
        fin_sc = fin[_N_TC:]
        recip = _tc_recip(fin_sc, _N_SC)
        parts.append(
            _make_sc_means(_N_SC)(fin_sc, hidden.reshape(-1), recip))
    means = parts[0] if len(parts) == 1 else jnp.concatenate(parts, axis=0)

    m2 = means.reshape(_N, _NB * _H)

    Wt = W.T
    b2 = b.reshape(1, _FC_OUT)
    return pl.pallas_call(
        _fc_step,
        grid=(_N // _BR,),
        in_specs=[
            pl.BlockSpec((_BR, _NB * _H), lambda i: (i, 0)),
            pl.BlockSpec((_NB * _H, _FC_OUT), lambda i: (0, 0)),
            pl.BlockSpec((1, _FC_OUT), lambda i: (0, 0)),
        ],
        out_specs=pl.BlockSpec((_BR, _FC_OUT), lambda i: (i, 0)),
        out_shape=jax.ShapeDtypeStruct((_N, _FC_OUT), jnp.float32),
    )(m2, Wt, b2)


# speedup vs baseline: 6.2876x; 6.2876x over previous
import functools
import jax
import jax.numpy as jnp
import numpy as np
from jax import lax
from jax.experimental import pallas as pl
from jax.experimental.pallas import tpu as pltpu
from jax.experimental.pallas import tpu_sc as plsc

_NR = 8
_NW = 8
_H = 48
_NB = _NR * _NW
_N = 2048
_FC_OUT = 128
_BI = 16
_BR = 256

_N_TC = 1920
_N_SC = _N - _N_TC


def _pairwise_bins(ydash):
    r = jnp.linalg.norm(ydash[:, None, :] - ydash[None, :, :], axis=2)
    ring_ids = jnp.ceil((_NR - 1) * (jnp.log(r / 0.5) / 3.0))
    ring_ids = jnp.where(jnp.isneginf(ring_ids), 0.0, ring_ids)
    ring_ids = ring_ids.astype(jnp.int32)
    x_diff = ydash[:, 0] - ydash[:, 0][:, None]
    y_diff = ydash[:, 1] - ydash[:, 1][:, None]
    theta = jnp.arctan2(y_diff, x_diff)
    wedge_ids = theta * _NW / (2 * np.pi)
    wedge_ids = (wedge_ids + (_NW // 2 - 1)).astype(jnp.int32)
    fin = ring_ids * _NW + wedge_ids
    fin = jnp.where(fin < _NW, 0, fin)
    fin = jnp.where(fin >= _NW * _NW, 0, fin)
    return fin.astype(jnp.int32)


def _means_step(fin_ref, hid, m_ref):
    fin = fin_ref[...]
    k_iota = jax.lax.broadcasted_iota(jnp.int32, (_BI, _NB, _N), 1)
    onehot = (fin[:, None, :] == k_iota + _NW).astype(jnp.float32)
    onehot = onehot.reshape(_BI * _NB, _N)

    s = jnp.dot(onehot, hid[...], preferred_element_type=jnp.float32)
    cnt = s[:, _H:_H + 1]
    recip = 1.0 / jnp.maximum(cnt, 1.0)
    m_ref[...] = s[:, :_H] * recip


def _tc_means(fin_tc, hidden):
    hidden_aug = jnp.concatenate(
        [hidden, jnp.ones((_N, 1), hidden.dtype),
         jnp.zeros((_N, 64 - _H - 1), hidden.dtype)], axis=1)
    return pl.pallas_call(
        _means_step,
        grid=(_N_TC // _BI,),
        in_specs=[
            pl.BlockSpec((_BI, _N), lambda i: (i, 0)),
            pl.BlockSpec((_N, 64), lambda i: (0, 0)),
        ],
        out_specs=pl.BlockSpec((_BI * _NB, _H), lambda i: (i, 0)),
        out_shape=jax.ShapeDtypeStruct((_N_TC * _NB, _H), jnp.float32),
    )(fin_tc, hidden_aug)


def _recip_step(fin_ref, r_ref):
    fin = fin_ref[...]
    k_iota = jax.lax.broadcasted_iota(jnp.int32, (_BI, _NB, _N), 1)
    cnt = jnp.sum((fin[:, None, :] == k_iota + _NW).astype(jnp.float32),
                  axis=2)
    r_ref[...] = 1.0 / jnp.maximum(cnt, 1.0)


def _tc_recip(fin_sc, nsc):
    return pl.pallas_call(
        _recip_step,
        grid=(nsc // _BI,),
        in_specs=[pl.BlockSpec((_BI, _N), lambda i: (i, 0))],
        out_specs=pl.BlockSpec((_BI, _NB), lambda i: (i, 0)),
        out_shape=jax.ShapeDtypeStruct((nsc, _NB), jnp.float32),
    )(fin_sc)


def _make_sc_means(nsc):
    ag_per_w = nsc // 32
    mesh = plsc.VectorSubcoreMesh(core_axis_name="c", subcore_axis_name="s")

    @functools.partial(
        pl.kernel,
        mesh=mesh,
        compiler_params=pltpu.CompilerParams(use_tc_tiling_on_sc=False),
        out_type=jax.ShapeDtypeStruct((nsc * _NB, _H), jnp.float32),
        scratch_types=[
            pltpu.VMEM((_N * _H,), jnp.float32),
            pltpu.VMEM((_N,), jnp.int32),
            pltpu.VMEM((4, _NB * _H), jnp.float32),
            pltpu.VMEM((_NB, _H), jnp.float32),
            pltpu.VMEM((_NB,), jnp.float32),
            pltpu.SemaphoreType.DMA,
        ],
    )
    def sc_means(fin_hbm, hidden1d_hbm, recip_hbm, out_hbm,
                 hid_v, fin_v, acc_v, out_v, recip_v, sem):
        cid = lax.axis_index("c")
        sid = lax.axis_index("s")
        wid = cid * 16 + sid

        pltpu.sync_copy(hidden1d_hbm, hid_v)

        z16 = jnp.zeros((16,), jnp.float32)
        for row in range(_NB):
            for c3 in range(_H // 16):
                out_v[row, pl.ds(c3 * 16, 16)] = z16

        def agent_body(a, _):
            agent = wid * ag_per_w + a
            pltpu.sync_copy(fin_hbm.at[agent], fin_v)
            pltpu.sync_copy(recip_hbm.at[agent], recip_v)
            for p in range(4):
                for t in range(_NB * _H // 16):
                    acc_v[p, pl.ds(t * 16, 16)] = z16

            def j_body(g, _):
                j0 = g * 16
                idxv = fin_v[pl.ds(j0, 16)] * _H
                for u in range(16):
                    off = idxv[u]
                    src = (j0 + u) * _H
                    for c3 in range(_H // 16):
                        sl = pl.ds(off + c3 * 16, 16)
                        acc_v[u % 4, sl] = (
                            acc_v[u % 4, sl]
                            + hid_v[pl.ds(src + c3 * 16, 16)])
                return 0

            lax.fori_loop(0, _N // 16, j_body, 0)

            for k in range(_NW, _NB):
                slot = k - _NW
                rvec = recip_v[pl.ds((slot // 16) * 16, 16)]
                rv = lax.gather(
                    rvec, jnp.full((16, 1), slot % 16, jnp.int32),
                    lax.GatherDimensionNumbers(
                        offset_dims=(), collapsed_slice_dims=(0,),
                        start_index_map=(0,)),
                    slice_sizes=(1,),
                    mode=lax.GatherScatterMode.PROMISE_IN_BOUNDS)
                for c3 in range(_H // 16):
                    sl = pl.ds(k * _H + c3 * 16, 16)
                    tot = ((acc_v[0, sl] + acc_v[1, sl])
                           + (acc_v[2, sl] + acc_v[3, sl]))
                    out_v[k - _NW, pl.ds(c3 * 16, 16)] = tot * rv

            pltpu.sync_copy(out_v, out_hbm.at[pl.ds(agent * _NB, _NB)])
            return 0

        lax.fori_loop(0, ag_per_w, agent_body, 0)

    return sc_means


def _fc_step(m, Wt, b, out_ref):
    o = jnp.dot(m[...], Wt[...], preferred_element_type=jnp.float32) + b[...]
    out_ref[...] = jnp.maximum(o, 0.0)


def kernel(y_pred, x_start, hidden, W, b):
    del x_start
    fin = _pairwise_bins(jax.lax.stop_gradient(y_pred))

    parts = []
    if _N_TC:
        parts.append(_tc_means(fin[:_N_TC], hidden))
    if _N_SC:
        fin_sc = fin[_N_TC:]
        recip = _tc_recip(fin_sc, _N_SC)
        parts.append(
            _make_sc_means(_N_SC)(fin_sc, hidden.reshape(-1), recip))
    means = parts[0] if len(parts) == 1 else jnp.concatenate(parts, axis=0)

    m2 = means.reshape(_N, _NB * _H)

    Wt = W.T
    b2 = b.reshape(1, _FC_OUT)
    return pl.pallas_call(
        _fc_step,
        grid=(_N // _BR,),
        in_specs=[
            pl.BlockSpec((_BR, _NB * _H), lambda i: (i, 0)),
            pl.BlockSpec((_NB * _H, _FC_OUT), lambda i: (0, 0)),
            pl.BlockSpec((1, _FC_OUT), lambda i: (0, 0)),
        ],
        out_specs=pl.BlockSpec((_BR, _FC_OUT), lambda i: (i, 0)),
        out_shape=jax.ShapeDtypeStruct((_N, _FC_OUT), jnp.float32),
    )(m2, Wt, b2)


# speedup vs baseline: 8.0635x; 1.2824x over previous
import functools
import jax
import jax.numpy as jnp
import numpy as np
from jax import lax
from jax.experimental import pallas as pl
from jax.experimental.pallas import tpu as pltpu
from jax.experimental.pallas import tpu_sc as plsc

_NR = 8
_NW = 8
_H = 48
_NB = _NR * _NW
_N = 2048
_FC_OUT = 128
_BI = 16
_BR = 256

_N_TC = _N
_N_SC = _N - _N_TC


def _pairwise_bins(ydash):
    r = jnp.linalg.norm(ydash[:, None, :] - ydash[None, :, :], axis=2)
    ring_ids = jnp.ceil((_NR - 1) * (jnp.log(r / 0.5) / 3.0))
    ring_ids = jnp.where(jnp.isneginf(ring_ids), 0.0, ring_ids)
    ring_ids = ring_ids.astype(jnp.int32)
    x_diff = ydash[:, 0] - ydash[:, 0][:, None]
    y_diff = ydash[:, 1] - ydash[:, 1][:, None]
    theta = jnp.arctan2(y_diff, x_diff)
    wedge_ids = theta * _NW / (2 * np.pi)
    wedge_ids = (wedge_ids + (_NW // 2 - 1)).astype(jnp.int32)
    fin = ring_ids * _NW + wedge_ids
    fin = jnp.where(fin < _NW, 0, fin)
    fin = jnp.where(fin >= _NW * _NW, 0, fin)
    return fin.astype(jnp.int32)


def _means_step(fin_ref, hid, m_ref):
    fin = fin_ref[...]
    k_iota = jax.lax.broadcasted_iota(jnp.int32, (_BI, _NB, _N), 1)
    onehot = (fin[:, None, :] == k_iota + _NW).astype(jnp.float32)
    onehot = onehot.reshape(_BI * _NB, _N)

    s = jnp.dot(onehot, hid[...], preferred_element_type=jnp.float32)
    cnt = s[:, _H:_H + 1]
    recip = 1.0 / jnp.maximum(cnt, 1.0)
    m_ref[...] = s[:, :_H] * recip


def _tc_means(fin_tc, hidden):
    hidden_aug = jnp.concatenate(
        [hidden, jnp.ones((_N, 1), hidden.dtype),
         jnp.zeros((_N, 64 - _H - 1), hidden.dtype)], axis=1)
    return pl.pallas_call(
        _means_step,
        grid=(_N_TC // _BI,),
        in_specs=[
            pl.BlockSpec((_BI, _N), lambda i: (i, 0)),
            pl.BlockSpec((_N, 64), lambda i: (0, 0)),
        ],
        out_specs=pl.BlockSpec((_BI * _NB, _H), lambda i: (i, 0)),
        out_shape=jax.ShapeDtypeStruct((_N_TC * _NB, _H), jnp.float32),
    )(fin_tc, hidden_aug)


def _recip_step(fin_ref, r_ref):
    fin = fin_ref[...]
    k_iota = jax.lax.broadcasted_iota(jnp.int32, (_BI, _NB, _N), 1)
    cnt = jnp.sum((fin[:, None, :] == k_iota + _NW).astype(jnp.float32),
                  axis=2)
    r_ref[...] = 1.0 / jnp.maximum(cnt, 1.0)


def _tc_recip(fin_sc, nsc):
    return pl.pallas_call(
        _recip_step,
        grid=(nsc // _BI,),
        in_specs=[pl.BlockSpec((_BI, _N), lambda i: (i, 0))],
        out_specs=pl.BlockSpec((_BI, _NB), lambda i: (i, 0)),
        out_shape=jax.ShapeDtypeStruct((nsc, _NB), jnp.float32),
    )(fin_sc)


def _make_sc_means(nsc):
    ag_per_w = nsc // 32
    mesh = plsc.VectorSubcoreMesh(core_axis_name="c", subcore_axis_name="s")

    @functools.partial(
        pl.kernel,
        mesh=mesh,
        compiler_params=pltpu.CompilerParams(use_tc_tiling_on_sc=False),
        out_type=jax.ShapeDtypeStruct((nsc * _NB, _H), jnp.float32),
        scratch_types=[
            pltpu.VMEM((_N * _H,), jnp.float32),
            pltpu.VMEM((_N,), jnp.int32),
            pltpu.VMEM((4, _NB * _H), jnp.float32),
            pltpu.VMEM((_NB, _H), jnp.float32),
            pltpu.VMEM((_NB,), jnp.float32),
            pltpu.SemaphoreType.DMA,
        ],
    )
    def sc_means(fin_hbm, hidden1d_hbm, recip_hbm, out_hbm,
                 hid_v, fin_v, acc_v, out_v, recip_v, sem):
        cid = lax.axis_index("c")
        sid = lax.axis_index("s")
        wid = cid * 16 + sid

        pltpu.sync_copy(hidden1d_hbm, hid_v)

        z16 = jnp.zeros((16,), jnp.float32)
        for row in range(_NB):
            for c3 in range(_H // 16):
                out_v[row, pl.ds(c3 * 16, 16)] = z16

        def agent_body(a, _):
            agent = wid * ag_per_w + a
            pltpu.sync_copy(fin_hbm.at[agent], fin_v)
            pltpu.sync_copy(recip_hbm.at[agent], recip_v)
            for p in range(4):
                for t in range(_NB * _H // 16):
                    acc_v[p, pl.ds(t * 16, 16)] = z16

            def j_body(g, _):
                j0 = g * 16
                idxv = fin_v[pl.ds(j0, 16)] * _H
                for u in range(16):
                    off = idxv[u]
                    src = (j0 + u) * _H
                    for c3 in range(_H // 16):
                        sl = pl.ds(off + c3 * 16, 16)
                        acc_v[u % 4, sl] = (
                            acc_v[u % 4, sl]
                            + hid_v[pl.ds(src + c3 * 16, 16)])
                return 0

            lax.fori_loop(0, _N // 16, j_body, 0)

            for k in range(_NW, _NB):
                slot = k - _NW
                rvec = recip_v[pl.ds((slot // 16) * 16, 16)]
                rv = lax.gather(
                    rvec, jnp.full((16, 1), slot % 16, jnp.int32),
                    lax.GatherDimensionNumbers(
                        offset_dims=(), collapsed_slice_dims=(0,),
                        start_index_map=(0,)),
                    slice_sizes=(1,),
                    mode=lax.GatherScatterMode.PROMISE_IN_BOUNDS)
                for c3 in range(_H // 16):
                    sl = pl.ds(k * _H + c3 * 16, 16)
                    tot = ((acc_v[0, sl] + acc_v[1, sl])
                           + (acc_v[2, sl] + acc_v[3, sl]))
                    out_v[k - _NW, pl.ds(c3 * 16, 16)] = tot * rv

            pltpu.sync_copy(out_v, out_hbm.at[pl.ds(agent * _NB, _NB)])
            return 0

        lax.fori_loop(0, ag_per_w, agent_body, 0)

    return sc_means


def _fc_step(m, Wt, b, out_ref):
    o = jnp.dot(m[...], Wt[...], preferred_element_type=jnp.float32) + b[...]
    out_ref[...] = jnp.maximum(o, 0.0)


def kernel(y_pred, x_start, hidden, W, b):
    del x_start
    fin = _pairwise_bins(jax.lax.stop_gradient(y_pred))

    parts = []
    if _N_TC:
        parts.append(_tc_means(fin[:_N_TC], hidden))
    if _N_SC:
        fin_sc = fin[_N_TC:]
        recip = _tc_recip(fin_sc, _N_SC)
        parts.append(
            _make_sc_means(_N_SC)(fin_sc, hidden.reshape(-1), recip))
    means = parts[0] if len(parts) == 1 else jnp.concatenate(parts, axis=0)

    m2 = means.reshape(_N, _NB * _H)

    Wt = W.T
    b2 = b.reshape(1, _FC_OUT)
    return pl.pallas_call(
        _fc_step,
        grid=(_N // _BR,),
        in_specs=[
            pl.BlockSpec((_BR, _NB * _H), lambda i: (i, 0)),
            pl.BlockSpec((_NB * _H, _FC_OUT), lambda i: (0, 0)),
            pl.BlockSpec((1, _FC_OUT), lambda i: (0, 0)),
        ],
        out_specs=pl.BlockSpec((_BR, _FC_OUT), lambda i: (i, 0)),
        out_shape=jax.ShapeDtypeStruct((_N, _FC_OUT), jnp.float32),
    )(m2, Wt, b2)


# speedup vs baseline: 8.5257x; 1.0573x over previous
import functools
import jax
import jax.numpy as jnp
import numpy as np
from jax import lax
from jax.experimental import pallas as pl
from jax.experimental.pallas import tpu as pltpu
from jax.experimental.pallas import tpu_sc as plsc

_NR = 8
_NW = 8
_H = 48
_NB = _NR * _NW
_N = 2048
_FC_OUT = 128
_BI = 32
_BR = 256

_N_TC = _N
_N_SC = _N - _N_TC


def _pairwise_bins(ydash):
    r = jnp.linalg.norm(ydash[:, None, :] - ydash[None, :, :], axis=2)
    ring_ids = jnp.ceil((_NR - 1) * (jnp.log(r / 0.5) / 3.0))
    ring_ids = jnp.where(jnp.isneginf(ring_ids), 0.0, ring_ids)
    ring_ids = ring_ids.astype(jnp.int32)
    x_diff = ydash[:, 0] - ydash[:, 0][:, None]
    y_diff = ydash[:, 1] - ydash[:, 1][:, None]
    theta = jnp.arctan2(y_diff, x_diff)
    wedge_ids = theta * _NW / (2 * np.pi)
    wedge_ids = (wedge_ids + (_NW // 2 - 1)).astype(jnp.int32)
    fin = ring_ids * _NW + wedge_ids
    fin = jnp.where(fin < _NW, 0, fin)
    fin = jnp.where(fin >= _NW * _NW, 0, fin)
    return fin.astype(jnp.int32)


def _means_step(fin_ref, hid, m_ref):
    fin = fin_ref[...]
    k_iota = jax.lax.broadcasted_iota(jnp.int32, (_BI, _NB, _N), 1)
    onehot = (fin[:, None, :] == k_iota + _NW).astype(jnp.float32)
    onehot = onehot.reshape(_BI * _NB, _N)

    s = jnp.dot(onehot, hid[...], preferred_element_type=jnp.float32)
    cnt = s[:, _H:_H + 1]
    recip = 1.0 / jnp.maximum(cnt, 1.0)
    m_ref[...] = s[:, :_H] * recip


def _tc_means(fin_tc, hidden):
    hidden_aug = jnp.concatenate(
        [hidden, jnp.ones((_N, 1), hidden.dtype),
         jnp.zeros((_N, 64 - _H - 1), hidden.dtype)], axis=1)
    return pl.pallas_call(
        _means_step,
        grid=(_N_TC // _BI,),
        in_specs=[
            pl.BlockSpec((_BI, _N), lambda i: (i, 0)),
            pl.BlockSpec((_N, 64), lambda i: (0, 0)),
        ],
        out_specs=pl.BlockSpec((_BI * _NB, _H), lambda i: (i, 0)),
        out_shape=jax.ShapeDtypeStruct((_N_TC * _NB, _H), jnp.float32),
    )(fin_tc, hidden_aug)


def _recip_step(fin_ref, r_ref):
    fin = fin_ref[...]
    k_iota = jax.lax.broadcasted_iota(jnp.int32, (_BI, _NB, _N), 1)
    cnt = jnp.sum((fin[:, None, :] == k_iota + _NW).astype(jnp.float32),
                  axis=2)
    r_ref[...] = 1.0 / jnp.maximum(cnt, 1.0)


def _tc_recip(fin_sc, nsc):
    return pl.pallas_call(
        _recip_step,
        grid=(nsc // _BI,),
        in_specs=[pl.BlockSpec((_BI, _N), lambda i: (i, 0))],
        out_specs=pl.BlockSpec((_BI, _NB), lambda i: (i, 0)),
        out_shape=jax.ShapeDtypeStruct((nsc, _NB), jnp.float32),
    )(fin_sc)


def _make_sc_means(nsc):
    ag_per_w = nsc // 32
    mesh = plsc.VectorSubcoreMesh(core_axis_name="c", subcore_axis_name="s")

    @functools.partial(
        pl.kernel,
        mesh=mesh,
        compiler_params=pltpu.CompilerParams(use_tc_tiling_on_sc=False),
        out_type=jax.ShapeDtypeStruct((nsc * _NB, _H), jnp.float32),
        scratch_types=[
            pltpu.VMEM((_N * _H,), jnp.float32),
            pltpu.VMEM((_N,), jnp.int32),
            pltpu.VMEM((4, _NB * _H), jnp.float32),
            pltpu.VMEM((_NB, _H), jnp.float32),
            pltpu.VMEM((_NB,), jnp.float32),
            pltpu.SemaphoreType.DMA,
        ],
    )
    def sc_means(fin_hbm, hidden1d_hbm, recip_hbm, out_hbm,
                 hid_v, fin_v, acc_v, out_v, recip_v, sem):
        cid = lax.axis_index("c")
        sid = lax.axis_index("s")
        wid = cid * 16 + sid

        pltpu.sync_copy(hidden1d_hbm, hid_v)

        z16 = jnp.zeros((16,), jnp.float32)
        for row in range(_NB):
            for c3 in range(_H // 16):
                out_v[row, pl.ds(c3 * 16, 16)] = z16

        def agent_body(a, _):
            agent = wid * ag_per_w + a
            pltpu.sync_copy(fin_hbm.at[agent], fin_v)
            pltpu.sync_copy(recip_hbm.at[agent], recip_v)
            for p in range(4):
                for t in range(_NB * _H // 16):
                    acc_v[p, pl.ds(t * 16, 16)] = z16

            def j_body(g, _):
                j0 = g * 16
                idxv = fin_v[pl.ds(j0, 16)] * _H
                for u in range(16):
                    off = idxv[u]
                    src = (j0 + u) * _H
                    for c3 in range(_H // 16):
                        sl = pl.ds(off + c3 * 16, 16)
                        acc_v[u % 4, sl] = (
                            acc_v[u % 4, sl]
                            + hid_v[pl.ds(src + c3 * 16, 16)])
                return 0

            lax.fori_loop(0, _N // 16, j_body, 0)

            for k in range(_NW, _NB):
                slot = k - _NW
                rvec = recip_v[pl.ds((slot // 16) * 16, 16)]
                rv = lax.gather(
                    rvec, jnp.full((16, 1), slot % 16, jnp.int32),
                    lax.GatherDimensionNumbers(
                        offset_dims=(), collapsed_slice_dims=(0,),
                        start_index_map=(0,)),
                    slice_sizes=(1,),
                    mode=lax.GatherScatterMode.PROMISE_IN_BOUNDS)
                for c3 in range(_H // 16):
                    sl = pl.ds(k * _H + c3 * 16, 16)
                    tot = ((acc_v[0, sl] + acc_v[1, sl])
                           + (acc_v[2, sl] + acc_v[3, sl]))
                    out_v[k - _NW, pl.ds(c3 * 16, 16)] = tot * rv

            pltpu.sync_copy(out_v, out_hbm.at[pl.ds(agent * _NB, _NB)])
            return 0

        lax.fori_loop(0, ag_per_w, agent_body, 0)

    return sc_means


def _fc_step(m, Wt, b, out_ref):
    o = jnp.dot(m[...], Wt[...], preferred_element_type=jnp.float32) + b[...]
    out_ref[...] = jnp.maximum(o, 0.0)


def kernel(y_pred, x_start, hidden, W, b):
    del x_start
    fin = _pairwise_bins(jax.lax.stop_gradient(y_pred))

    parts = []
    if _N_TC:
        parts.append(_tc_means(fin[:_N_TC], hidden))
    if _N_SC:
        fin_sc = fin[_N_TC:]
        recip = _tc_recip(fin_sc, _N_SC)
        parts.append(
            _make_sc_means(_N_SC)(fin_sc, hidden.reshape(-1), recip))
    means = parts[0] if len(parts) == 1 else jnp.concatenate(parts, axis=0)

    m2 = means.reshape(_N, _NB * _H)

    Wt = W.T
    b2 = b.reshape(1, _FC_OUT)
    return pl.pallas_call(
        _fc_step,
        grid=(_N // _BR,),
        in_specs=[
            pl.BlockSpec((_BR, _NB * _H), lambda i: (i, 0)),
            pl.BlockSpec((_NB * _H, _FC_OUT), lambda i: (0, 0)),
            pl.BlockSpec((1, _FC_OUT), lambda i: (0, 0)),
        ],
        out_specs=pl.BlockSpec((_BR, _FC_OUT), lambda i: (i, 0)),
        out_shape=jax.ShapeDtypeStruct((_N, _FC_OUT), jnp.float32),
    )(m2, Wt, b2)


# speedup vs baseline: 8.7608x; 1.0276x over previous
import functools
import jax
import jax.numpy as jnp
import numpy as np
from jax import lax
from jax.experimental import pallas as pl
from jax.experimental.pallas import tpu as pltpu
from jax.experimental.pallas import tpu_sc as plsc

_NR = 8
_NW = 8
_H = 48
_NB = _NR * _NW
_N = 2048
_FC_OUT = 128
_BI = 64
_BR = 256

_N_TC = _N
_N_SC = _N - _N_TC


def _pairwise_bins(ydash):
    r = jnp.linalg.norm(ydash[:, None, :] - ydash[None, :, :], axis=2)
    ring_ids = jnp.ceil((_NR - 1) * (jnp.log(r / 0.5) / 3.0))
    ring_ids = jnp.where(jnp.isneginf(ring_ids), 0.0, ring_ids)
    ring_ids = ring_ids.astype(jnp.int32)
    x_diff = ydash[:, 0] - ydash[:, 0][:, None]
    y_diff = ydash[:, 1] - ydash[:, 1][:, None]
    theta = jnp.arctan2(y_diff, x_diff)
    wedge_ids = theta * _NW / (2 * np.pi)
    wedge_ids = (wedge_ids + (_NW // 2 - 1)).astype(jnp.int32)
    fin = ring_ids * _NW + wedge_ids
    fin = jnp.where(fin < _NW, 0, fin)
    fin = jnp.where(fin >= _NW * _NW, 0, fin)
    return fin.astype(jnp.int32)


def _means_step(fin_ref, hid, m_ref):
    fin = fin_ref[...]
    k_iota = jax.lax.broadcasted_iota(jnp.int32, (_BI, _NB, _N), 1)
    onehot = (fin[:, None, :] == k_iota + _NW).astype(jnp.float32)
    onehot = onehot.reshape(_BI * _NB, _N)

    s = jnp.dot(onehot, hid[...], preferred_element_type=jnp.float32)
    cnt = s[:, _H:_H + 1]
    recip = 1.0 / jnp.maximum(cnt, 1.0)
    m_ref[...] = s[:, :_H] * recip


def _tc_means(fin_tc, hidden):
    hidden_aug = jnp.concatenate(
        [hidden, jnp.ones((_N, 1), hidden.dtype),
         jnp.zeros((_N, 64 - _H - 1), hidden.dtype)], axis=1)
    return pl.pallas_call(
        _means_step,
        grid=(_N_TC // _BI,),
        in_specs=[
            pl.BlockSpec((_BI, _N), lambda i: (i, 0)),
            pl.BlockSpec((_N, 64), lambda i: (0, 0)),
        ],
        out_specs=pl.BlockSpec((_BI * _NB, _H), lambda i: (i, 0)),
        out_shape=jax.ShapeDtypeStruct((_N_TC * _NB, _H), jnp.float32),
    )(fin_tc, hidden_aug)


def _recip_step(fin_ref, r_ref):
    fin = fin_ref[...]
    k_iota = jax.lax.broadcasted_iota(jnp.int32, (_BI, _NB, _N), 1)
    cnt = jnp.sum((fin[:, None, :] == k_iota + _NW).astype(jnp.float32),
                  axis=2)
    r_ref[...] = 1.0 / jnp.maximum(cnt, 1.0)


def _tc_recip(fin_sc, nsc):
    return pl.pallas_call(
        _recip_step,
        grid=(nsc // _BI,),
        in_specs=[pl.BlockSpec((_BI, _N), lambda i: (i, 0))],
        out_specs=pl.BlockSpec((_BI, _NB), lambda i: (i, 0)),
        out_shape=jax.ShapeDtypeStruct((nsc, _NB), jnp.float32),
    )(fin_sc)


def _make_sc_means(nsc):
    ag_per_w = nsc // 32
    mesh = plsc.VectorSubcoreMesh(core_axis_name="c", subcore_axis_name="s")

    @functools.partial(
        pl.kernel,
        mesh=mesh,
        compiler_params=pltpu.CompilerParams(use_tc_tiling_on_sc=False),
        out_type=jax.ShapeDtypeStruct((nsc * _NB, _H), jnp.float32),
        scratch_types=[
            pltpu.VMEM((_N * _H,), jnp.float32),
            pltpu.VMEM((_N,), jnp.int32),
            pltpu.VMEM((4, _NB * _H), jnp.float32),
            pltpu.VMEM((_NB, _H), jnp.float32),
            pltpu.VMEM((_NB,), jnp.float32),
            pltpu.SemaphoreType.DMA,
        ],
    )
    def sc_means(fin_hbm, hidden1d_hbm, recip_hbm, out_hbm,
                 hid_v, fin_v, acc_v, out_v, recip_v, sem):
        cid = lax.axis_index("c")
        sid = lax.axis_index("s")
        wid = cid * 16 + sid

        pltpu.sync_copy(hidden1d_hbm, hid_v)

        z16 = jnp.zeros((16,), jnp.float32)
        for row in range(_NB):
            for c3 in range(_H // 16):
                out_v[row, pl.ds(c3 * 16, 16)] = z16

        def agent_body(a, _):
            agent = wid * ag_per_w + a
            pltpu.sync_copy(fin_hbm.at[agent], fin_v)
            pltpu.sync_copy(recip_hbm.at[agent], recip_v)
            for p in range(4):
                for t in range(_NB * _H // 16):
                    acc_v[p, pl.ds(t * 16, 16)] = z16

            def j_body(g, _):
                j0 = g * 16
                idxv = fin_v[pl.ds(j0, 16)] * _H
                for u in range(16):
                    off = idxv[u]
                    src = (j0 + u) * _H
                    for c3 in range(_H // 16):
                        sl = pl.ds(off + c3 * 16, 16)
                        acc_v[u % 4, sl] = (
                            acc_v[u % 4, sl]
                            + hid_v[pl.ds(src + c3 * 16, 16)])
                return 0

            lax.fori_loop(0, _N // 16, j_body, 0)

            for k in range(_NW, _NB):
                slot = k - _NW
                rvec = recip_v[pl.ds((slot // 16) * 16, 16)]
                rv = lax.gather(
                    rvec, jnp.full((16, 1), slot % 16, jnp.int32),
                    lax.GatherDimensionNumbers(
                        offset_dims=(), collapsed_slice_dims=(0,),
                        start_index_map=(0,)),
                    slice_sizes=(1,),
                    mode=lax.GatherScatterMode.PROMISE_IN_BOUNDS)
                for c3 in range(_H // 16):
                    sl = pl.ds(k * _H + c3 * 16, 16)
                    tot = ((acc_v[0, sl] + acc_v[1, sl])
                           + (acc_v[2, sl] + acc_v[3, sl]))
                    out_v[k - _NW, pl.ds(c3 * 16, 16)] = tot * rv

            pltpu.sync_copy(out_v, out_hbm.at[pl.ds(agent * _NB, _NB)])
            return 0

        lax.fori_loop(0, ag_per_w, agent_body, 0)

    return sc_means


def _fc_step(m, Wt, b, out_ref):
    o = jnp.dot(m[...], Wt[...], preferred_element_type=jnp.float32) + b[...]
    out_ref[...] = jnp.maximum(o, 0.0)


def kernel(y_pred, x_start, hidden, W, b):
    del x_start
    fin = _pairwise_bins(jax.lax.stop_gradient(y_pred))

    parts = []
    if _N_TC:
        parts.append(_tc_means(fin[:_N_TC], hidden))
    if _N_SC:
        fin_sc = fin[_N_TC:]
        recip = _tc_recip(fin_sc, _N_SC)
        parts.append(
            _make_sc_means(_N_SC)(fin_sc, hidden.reshape(-1), recip))
    means = parts[0] if len(parts) == 1 else jnp.concatenate(parts, axis=0)

    m2 = means.reshape(_N, _NB * _H)

    Wt = W.T
    b2 = b.reshape(1, _FC_OUT)
    return pl.pallas_call(
        _fc_step,
        grid=(_N // _BR,),
        in_specs=[
            pl.BlockSpec((_BR, _NB * _H), lambda i: (i, 0)),
            pl.BlockSpec((_NB * _H, _FC_OUT), lambda i: (0, 0)),
            pl.BlockSpec((1, _FC_OUT), lambda i: (0, 0)),
        ],
        out_specs=pl.BlockSpec((_BR, _FC_OUT), lambda i: (i, 0)),
        out_shape=jax.ShapeDtypeStruct((_N, _FC_OUT), jnp.float32),
    )(m2, Wt, b2)
